# Initial kernel scaffold; baseline (speedup 1.0000x reference)
import jax
import jax.numpy as jnp
from jax.experimental import pallas as pl


def kernel(x, pe, Lambda, edge_index, edge_attr, batch, mlp_W1, mlp_b1, mlp_g1, mlp_bt1, mlp_W2, mlp_b2, pe_lin_W, pe_lin_b, pe_edge_W, pe_edge_b, enc_Wq, enc_bq, enc_out_W, enc_out_b, norm_g, norm_b):
    raise NotImplementedError("write your pallas kernel here")


# speedup vs baseline: 7.0123x; 7.0123x over previous
import functools

import jax
import jax.numpy as jnp
from jax import lax
from jax.experimental import pallas as pl
from jax.experimental.pallas import tpu as pltpu
from jax.experimental.pallas import tpu_sc as plsc

N = 10000
H = 128
HPE = 16
P = 8
Q = 4
G = 16
E = 160000
L = 3

W = 128
NBLK = E // W
NW = 32
NS = 16
N2 = 10240
ROWS_W = N2 // NW
GW = 64

BE = 1280
NBE = E // BE


def _nblocks_for(wid, nworkers):
    full = NBLK // nworkers
    extra = NBLK % nworkers
    return jnp.where(wid < extra, full + 1, full).astype(jnp.int32)


def _sc_edge_gather_body(z2_hbm, src_hbm, dst_hbm, batch_hbm,
                    prod_hbm, bsrc_hbm,
                    src_v, dst_v, zs_v, zd_v, prod_v, bsrc_v,
                    sem0, sem1, sem2):
    wid = (lax.axis_index("s") * 2 + lax.axis_index("c")).astype(jnp.int32)
    nb = _nblocks_for(wid, NW)

    def block_body(g, _):
        blk = wid + g * NW
        e0 = blk * W
        pltpu.sync_copy(src_hbm.at[pl.ds(e0, W)], src_v)
        pltpu.sync_copy(dst_hbm.at[pl.ds(e0, W)], dst_v)
        cp0 = pltpu.async_copy(z2_hbm.at[src_v], zs_v, sem0)
        cp1 = pltpu.async_copy(z2_hbm.at[dst_v], zd_v, sem1)
        cpb = pltpu.async_copy(batch_hbm.at[src_v], bsrc_v, sem2)
        cp0.wait()
        cp1.wait()
        cpb.wait()

        def edge_body(e, _):
            for c in range(H // 16):
                sl = pl.ds(c * 16, 16)
                prod_v[e, sl] = zs_v[e, sl] * zd_v[e, sl]
            return 0

        lax.fori_loop(0, W, edge_body, 0)
        pltpu.sync_copy(prod_v, prod_hbm.at[pl.ds(e0, W)])
        pltpu.sync_copy(bsrc_v, bsrc_hbm.at[pl.ds(e0, W)])
        return 0

    lax.fori_loop(0, nb, block_body, 0)


def _make_sc_scatter_body(gate):
    def body(table_hbm, pay_hbm, pi_hbm, srcs_hbm, dsts_hbm, bnd_hbm, out_hbm,
             bnd_v, ids_v, srcw_v, dstw_v, pay_v, gat_v, acc_v,
             sem0, sem1, sem2):
        core = lax.axis_index("c").astype(jnp.int32)
        sid = lax.axis_index("s").astype(jnp.int32)
        w = sid * 2 + core
        lo = w * ROWS_W

        pltpu.sync_copy(bnd_hbm, bnd_v)
        s0 = bnd_v[pl.ds(w, 16)][0]
        s1 = bnd_v[pl.ds(w + 1, 16)][0]

        def zrow(r, _):
            for c in range(H // 16):
                acc_v[r, pl.ds(c * 16, 16)] = jnp.zeros((16,), jnp.float32)
            return 0

        lax.fori_loop(0, ROWS_W, zrow, 0)

        sA = pl.multiple_of((s0 // GW) * GW, GW)
        nwin = (s1 - sA + GW - 1) // GW

        def win_body(j, _):
            base = pl.multiple_of(sA + j * GW, GW)
            cpi = pltpu.async_copy(pi_hbm.at[pl.ds(base, GW)], ids_v, sem0)
            cps = pltpu.async_copy(srcs_hbm.at[pl.ds(base, GW)], srcw_v, sem1)
            cpd = pltpu.async_copy(dsts_hbm.at[pl.ds(base, GW)], dstw_v, sem2)
            cpi.wait()
            cpp = pltpu.async_copy(pay_hbm.at[ids_v], pay_v, sem0)
            cps.wait()
            cpt = pltpu.async_copy(table_hbm.at[srcw_v], gat_v, sem1)
            cpd.wait()
            cpp.wait()
            cpt.wait()

            for c in range(GW // 16):
                dv = dstw_v[pl.ds(c * 16, 16)]
                for jj in range(16):
                    e = c * 16 + jj
                    g = base + e
                    r = dv[jj] - lo

                    @pl.when((g >= s0) & (g < s1))
                    def _():
                        for cc in range(H // 16):
                            sl = pl.ds(cc * 16, 16)
                            if gate:
                                acc_v[r, sl] = (acc_v[r, sl]
                                                + gat_v[e, sl] * pay_v[e, sl])
                            else:
                                acc_v[r, sl] = acc_v[r, sl] + jnp.maximum(
                                    gat_v[e, sl] + pay_v[e, sl], 0.0)
            return 0

        lax.fori_loop(0, nwin, win_body, 0)

        pltpu.sync_copy(acc_v, out_hbm.at[pl.ds(lo, ROWS_W)])

    return body


@functools.lru_cache(maxsize=None)
def _sc_kernels():
    mesh = plsc.VectorSubcoreMesh(
        core_axis_name="c", subcore_axis_name="s", num_cores=2, num_subcores=16)
    edge_gather = pl.kernel(
        _sc_edge_gather_body,
        out_type=(
            jax.ShapeDtypeStruct((E, H), jnp.float32),
            jax.ShapeDtypeStruct((E,), jnp.int32),
        ),
        mesh=mesh,
        scratch_types=[
            pltpu.VMEM((W,), jnp.int32),
            pltpu.VMEM((W,), jnp.int32),
            pltpu.VMEM((W, H), jnp.float32),
            pltpu.VMEM((W, H), jnp.float32),
            pltpu.VMEM((W, H), jnp.float32),
            pltpu.VMEM((W,), jnp.int32),
            pltpu.SemaphoreType.DMA,
            pltpu.SemaphoreType.DMA,
            pltpu.SemaphoreType.DMA,
        ],
    )
    def make_scatter(gate):
        payw = H
        return pl.kernel(
            _make_sc_scatter_body(gate),
            out_type=jax.ShapeDtypeStruct((N2, H), jnp.float32),
            mesh=mesh,
            scratch_types=[
                pltpu.VMEM((48,), jnp.int32),
                pltpu.VMEM((GW,), jnp.int32),
                pltpu.VMEM((GW,), jnp.int32),
                pltpu.VMEM((GW,), jnp.int32),
                pltpu.VMEM((GW, payw), jnp.float32),
                pltpu.VMEM((GW, H), jnp.float32),
                pltpu.VMEM((ROWS_W, H), jnp.float32),
                pltpu.SemaphoreType.DMA,
                pltpu.SemaphoreType.DMA,
                pltpu.SemaphoreType.DMA,
            ],
        )

    return edge_gather, make_scatter(False), make_scatter(True)


def _sc_edge_gather(z2, src, dst, batch):
    return _sc_kernels()[0](z2, src, dst, batch)


def _sc_scatter_relu(x, ea, pi, srcs, dsts, bnd):
    return _sc_kernels()[1](x, ea, pi, srcs, dsts, bnd)


def _sc_scatter_gate(z2, w, pi, srcs, dsts, bnd):
    return _sc_kernels()[2](z2, w, pi, srcs, dsts, bnd)


def _tc_edge_body(prod_ref, bsrc_ref, eattr_ref, lamr_ref, wq_ref, bq_ref,
                  wperm_ref, bo_ref, wpe_ref, bpe_ref, ea_ref, w_ref):
    eig = jnp.maximum(lamr_ref[...] * wq_ref[...] + bq_ref[...], 0.0)
    bs = bsrc_ref[0, 0, :]
    oh = (bs[:, None] == lax.broadcasted_iota(jnp.int32, (BE, G), 1))
    ee = jnp.dot(oh.astype(jnp.float32), eig, preferred_element_type=jnp.float32)
    prod = prod_ref[...]
    feats = []
    for q in range(Q):
        acc = prod[:, 0:HPE] * ee[:, q:q + 1]
        for p in range(1, P):
            acc = acc + prod[:, p * HPE:(p + 1) * HPE] * ee[:, p * Q + q:p * Q + q + 1]
        feats.append(acc)
    feat = jnp.concatenate(feats, axis=1)
    ea = (jnp.dot(feat, wperm_ref[...], preferred_element_type=jnp.float32)
          + bo_ref[...] + eattr_ref[...])
    ea_ref[...] = ea
    w = jax.nn.sigmoid(
        jnp.dot(ea, wpe_ref[...], preferred_element_type=jnp.float32) + bpe_ref[...])
    w_ref[...] = jnp.concatenate([w] * P, axis=1)


def _tc_edge(prod, bsrc3, edge_attr, lamr, wq, bq, wperm, bo, wpe, bpe):
    full = lambda s: pl.BlockSpec(s, lambda i: (0,) * len(s))
    return pl.pallas_call(
        _tc_edge_body,
        grid=(NBE,),
        in_specs=[
            pl.BlockSpec((BE, H), lambda i: (i, 0)),
            pl.BlockSpec((1, 1, BE), lambda i: (i, 0, 0)),
            pl.BlockSpec((BE, H), lambda i: (i, 0)),
            full((G, P * Q)),
            full((1, P * Q)),
            full((1, P * Q)),
            full((HPE * Q, H)),
            full((1, H)),
            full((H, HPE)),
            full((1, HPE)),
        ],
        out_specs=[
            pl.BlockSpec((BE, H), lambda i: (i, 0)),
            pl.BlockSpec((BE, H), lambda i: (i, 0)),
        ],
        out_shape=[
            jax.ShapeDtypeStruct((E, H), jnp.float32),
            jax.ShapeDtypeStruct((E, H), jnp.float32),
        ],
    )(prod, bsrc3, edge_attr, lamr, wq, bq, wperm, bo, wpe, bpe)


def _tc_node_x_body(last, x_ref, agg_ref, w1_ref, b1_ref,
                    g1_ref, bt1_ref, w2_ref, b2_ref, ng_ref, nb_ref, xo_ref):
    def bn(h, g, b):
        mu = jnp.mean(h, axis=0, keepdims=True)
        var = jnp.mean((h - mu) ** 2, axis=0, keepdims=True)
        return (h - mu) / jnp.sqrt(var + 1e-5) * g + b

    h = x_ref[...] + agg_ref[...]
    h = jnp.dot(h, w1_ref[...], preferred_element_type=jnp.float32) + b1_ref[...]
    h = jnp.maximum(bn(h, g1_ref[...], bt1_ref[...]), 0.0)
    xn = jnp.dot(h, w2_ref[...], preferred_element_type=jnp.float32) + b2_ref[...]
    xn = bn(xn, ng_ref[...], nb_ref[...])
    if not last:
        xn = jnp.maximum(xn, 0.0)
    xo_ref[...] = xn


def _tc_node_z_body(z2_ref, aggz_ref, plw_ref, plb_ref, zo_ref):
    hz = z2_ref[...] + aggz_ref[...]
    plw = plw_ref[...]
    cols = [
        jnp.dot(hz[:, p * HPE:(p + 1) * HPE], plw,
                preferred_element_type=jnp.float32) + plb_ref[...]
        for p in range(P)
    ]
    zo_ref[...] = jnp.concatenate(cols, axis=1)


def _tc_node(last, x, agg, z2, aggz,
             w1, b1, g1, bt1, w2, b2, plw, plb, ng, nb):
    xn = pl.pallas_call(
        functools.partial(_tc_node_x_body, last),
        out_shape=jax.ShapeDtypeStruct((N, H), jnp.float32),
    )(x, agg, w1, b1, g1, bt1, w2, b2, ng, nb)
    zn = pl.pallas_call(
        _tc_node_z_body,
        out_shape=jax.ShapeDtypeStruct((N, H), jnp.float32),
    )(z2, aggz, plw, plb)
    return xn, zn


def kernel(x, pe, Lambda, edge_index, edge_attr, batch, mlp_W1, mlp_b1,
           mlp_g1, mlp_bt1, mlp_W2, mlp_b2, pe_lin_W, pe_lin_b, pe_edge_W,
           pe_edge_b, enc_Wq, enc_bq, enc_out_W, enc_out_b, norm_g, norm_b):
    src = edge_index[0]
    dst = edge_index[1]
    pi = jnp.argsort(dst).astype(jnp.int32)
    dsts = dst[pi]
    srcs = src[pi]
    bnd = jnp.searchsorted(dsts, jnp.arange(NW + 1, dtype=jnp.int32) * ROWS_W,
                           ).astype(jnp.int32)
    bnd = jnp.concatenate([bnd, jnp.full((48 - NW - 1,), E, jnp.int32)])
    pad = jnp.zeros((GW,), jnp.int32)
    pi_p = jnp.concatenate([pi, pad])
    dsts_p = jnp.concatenate([dsts, pad])
    srcs_p = jnp.concatenate([srcs, pad])

    z2 = jnp.repeat(pe, HPE, axis=1)
    lamr = jnp.repeat(Lambda, Q, axis=1)

    x_cur = x
    for i in range(L):
        wq = jnp.tile(enc_Wq[i], P)[None, :]
        bq = jnp.tile(enc_bq[i], P)[None, :]
        wperm = enc_out_W[i].reshape(HPE, Q, H).transpose(1, 0, 2).reshape(HPE * Q, H)
        bo = enc_out_b[i][None, :]
        wpe = pe_edge_W[i]
        bpe = pe_edge_b[i][None, :]

        prod, bsrc = _sc_edge_gather(z2, src, dst, batch)
        bsrc3 = bsrc.reshape(NBE, 1, BE)
        ea, w = _tc_edge(prod, bsrc3, edge_attr, lamr, wq, bq, wperm, bo, wpe, bpe)
        agg = _sc_scatter_relu(x_cur, ea, pi_p, srcs_p, dsts_p, bnd)
        aggz = _sc_scatter_gate(z2, w, pi_p, srcs_p, dsts_p, bnd)
        x_cur, z2 = _tc_node(
            i == L - 1, x_cur, agg[:N], z2, aggz[:N],
            mlp_W1[i], mlp_b1[i][None, :], mlp_g1[i][None, :],
            mlp_bt1[i][None, :], mlp_W2[i], mlp_b2[i][None, :],
            pe_lin_W[i], pe_lin_b[i][None, :],
            norm_g[i][None, :], norm_b[i][None, :])

    z_out = z2.reshape(N, P, HPE).transpose(0, 2, 1)
    return x_cur, z_out


# speedup vs baseline: 8.6686x; 1.2362x over previous
import functools

import jax
import jax.numpy as jnp
from jax import lax
from jax.experimental import pallas as pl
from jax.experimental.pallas import tpu as pltpu
from jax.experimental.pallas import tpu_sc as plsc

N = 10000
H = 128
HPE = 16
P = 8
Q = 4
G = 16
E = 160000
L = 3

W = 128
NBLK = E // W
NW = 32
NS = 16
N2 = 10240
ROWS_W = N2 // NW
GW = 128

BE = 1280
NBE = E // BE


def _nblocks_for(wid, nworkers):
    full = NBLK // nworkers
    extra = NBLK % nworkers
    return jnp.where(wid < extra, full + 1, full).astype(jnp.int32)


def _sc_edge_gather_body(z2_hbm, src_hbm, dst_hbm, batch_hbm,
                    prod_hbm, bsrc_hbm,
                    src_v, dst_v, zs_v, zd_v, prod_v, bsrc_v,
                    sem0, sem1, sem2):
    wid = (lax.axis_index("s") * 2 + lax.axis_index("c")).astype(jnp.int32)
    nb = _nblocks_for(wid, NW)

    def block_body(g, _):
        blk = wid + g * NW
        e0 = blk * W
        pltpu.sync_copy(src_hbm.at[pl.ds(e0, W)], src_v)
        pltpu.sync_copy(dst_hbm.at[pl.ds(e0, W)], dst_v)
        cp0 = pltpu.async_copy(z2_hbm.at[src_v], zs_v, sem0)
        cp1 = pltpu.async_copy(z2_hbm.at[dst_v], zd_v, sem1)
        cpb = pltpu.async_copy(batch_hbm.at[src_v], bsrc_v, sem2)
        cp0.wait()
        cp1.wait()
        cpb.wait()

        def edge_body(e, _):
            for c in range(H // 16):
                sl = pl.ds(c * 16, 16)
                prod_v[e, sl] = zs_v[e, sl] * zd_v[e, sl]
            return 0

        lax.fori_loop(0, W, edge_body, 0)
        pltpu.sync_copy(prod_v, prod_hbm.at[pl.ds(e0, W)])
        pltpu.sync_copy(bsrc_v, bsrc_hbm.at[pl.ds(e0, W)])
        return 0

    lax.fori_loop(0, nb, block_body, 0)


def _make_sc_scatter_body(gate):
    def body(table_hbm, pay_hbm, pi_hbm, srcs_hbm, dsts_hbm, bnd_hbm, out_hbm,
             bnd_v, ids0, ids1, srw0, srw1, dsw0, dsw1, dhold,
             pay0, pay1, gat0, gat1, acc_v,
             si0, si1, sp0, sp1):
        core = lax.axis_index("c").astype(jnp.int32)
        sid = lax.axis_index("s").astype(jnp.int32)
        w = sid * 2 + core
        lo = w * ROWS_W

        pltpu.sync_copy(bnd_hbm, bnd_v)
        s0 = bnd_v[pl.ds(w, 16)][0]
        s1 = bnd_v[pl.ds(w + 1, 16)][0]

        def zrow(r, _):
            for c in range(H // 16):
                acc_v[r, pl.ds(c * 16, 16)] = jnp.zeros((16,), jnp.float32)
            return 0

        lax.fori_loop(0, ROWS_W, zrow, 0)

        sA = pl.multiple_of((s0 // GW) * GW, GW)
        nwin = (s1 - sA + GW - 1) // GW
        bufs = ((ids0, srw0, dsw0, si0, pay0, gat0, sp0),
                (ids1, srw1, dsw1, si1, pay1, gat1, sp1))

        def issue_idx(k, g):
            ids, srw, dsw, si = bufs[k][:4]
            base = pl.multiple_of(sA + g * GW, GW)
            pltpu.async_copy(pi_hbm.at[pl.ds(base, GW)], ids, si)
            pltpu.async_copy(srcs_hbm.at[pl.ds(base, GW)], srw, si)
            pltpu.async_copy(dsts_hbm.at[pl.ds(base, GW)], dsw, si)

        def wait_idx(k):
            ids, srw, dsw, si = bufs[k][:4]
            pltpu.make_async_copy(pi_hbm.at[pl.ds(0, GW)], ids, si).wait()
            pltpu.make_async_copy(pi_hbm.at[pl.ds(0, GW)], srw, si).wait()
            pltpu.make_async_copy(pi_hbm.at[pl.ds(0, GW)], dsw, si).wait()

        def issue_pay(k):
            ids, srw, _, _, pay, gat, sp = bufs[k]
            pltpu.async_copy(pay_hbm.at[ids], pay, sp)
            pltpu.async_copy(table_hbm.at[srw], gat, sp)

        def wait_pay(k):
            _, _, _, _, pay, gat, sp = bufs[k]
            pltpu.make_async_copy(pay_hbm.at[pl.ds(0, GW)], pay, sp).wait()
            pltpu.make_async_copy(pay_hbm.at[pl.ds(0, GW)], gat, sp).wait()

        def compute(k, g):
            _, _, _, _, pay, gat, _ = bufs[k]
            base = sA + g * GW

            def chunk(c, _):
                dv = dhold[pl.ds(c * 16, 16)]
                for jj in range(16):
                    gg = base + c * 16 + jj
                    r = dv[jj] - lo

                    @pl.when((gg >= s0) & (gg < s1))
                    def _():
                        def upd(cc, _):
                            e = c * 16 + jj
                            sl = pl.ds(cc * 16, 16)
                            if gate:
                                acc_v[r, sl] = (acc_v[r, sl]
                                                + gat[e, sl] * pay[e, sl])
                            else:
                                acc_v[r, sl] = acc_v[r, sl] + jnp.maximum(
                                    gat[e, sl] + pay[e, sl], 0.0)
                            return 0

                        lax.fori_loop(0, H // 16, upd, 0)
                return 0

            lax.fori_loop(0, GW // 16, chunk, 0)

        def hold_dst(k):
            dsw = bufs[k][2]
            for c in range(GW // 16):
                sl = pl.ds(c * 16, 16)
                dhold[sl] = dsw[sl]

        @pl.when(nwin >= 1)
        def _():
            issue_idx(0, 0)

        def phase(k, g):
            nk = 1 - k

            @pl.when(g < nwin)
            def _():
                wait_idx(k)
                issue_pay(k)

            @pl.when((g >= 1) & (g <= nwin))
            def _():
                wait_pay(nk)
                hold_dst(nk)

            @pl.when(g + 1 < nwin)
            def _():
                issue_idx(nk, g + 1)

            @pl.when((g >= 1) & (g <= nwin))
            def _():
                compute(nk, g - 1)

        def pair(g2, _):
            phase(0, g2 * 2)
            phase(1, g2 * 2 + 1)
            return 0

        lax.fori_loop(0, (nwin + 2) // 2, pair, 0)

        pltpu.sync_copy(acc_v, out_hbm.at[pl.ds(lo, ROWS_W)])

    return body


@functools.lru_cache(maxsize=None)
def _sc_kernels():
    mesh = plsc.VectorSubcoreMesh(
        core_axis_name="c", subcore_axis_name="s", num_cores=2, num_subcores=16)
    edge_gather = pl.kernel(
        _sc_edge_gather_body,
        out_type=(
            jax.ShapeDtypeStruct((E, H), jnp.float32),
            jax.ShapeDtypeStruct((E,), jnp.int32),
        ),
        mesh=mesh,
        scratch_types=[
            pltpu.VMEM((W,), jnp.int32),
            pltpu.VMEM((W,), jnp.int32),
            pltpu.VMEM((W, H), jnp.float32),
            pltpu.VMEM((W, H), jnp.float32),
            pltpu.VMEM((W, H), jnp.float32),
            pltpu.VMEM((W,), jnp.int32),
            pltpu.SemaphoreType.DMA,
            pltpu.SemaphoreType.DMA,
            pltpu.SemaphoreType.DMA,
        ],
    )
    def make_scatter(gate):
        payw = H
        return pl.kernel(
            _make_sc_scatter_body(gate),
            out_type=jax.ShapeDtypeStruct((N2, H), jnp.float32),
            mesh=mesh,
            scratch_types=[
                pltpu.VMEM((48,), jnp.int32),
                pltpu.VMEM((GW,), jnp.int32),
                pltpu.VMEM((GW,), jnp.int32),
                pltpu.VMEM((GW,), jnp.int32),
                pltpu.VMEM((GW,), jnp.int32),
                pltpu.VMEM((GW,), jnp.int32),
                pltpu.VMEM((GW,), jnp.int32),
                pltpu.VMEM((GW,), jnp.int32),
                pltpu.VMEM((GW, payw), jnp.float32),
                pltpu.VMEM((GW, payw), jnp.float32),
                pltpu.VMEM((GW, H), jnp.float32),
                pltpu.VMEM((GW, H), jnp.float32),
                pltpu.VMEM((ROWS_W, H), jnp.float32),
                pltpu.SemaphoreType.DMA,
                pltpu.SemaphoreType.DMA,
                pltpu.SemaphoreType.DMA,
                pltpu.SemaphoreType.DMA,
            ],
        )

    return edge_gather, make_scatter(False), make_scatter(True)


def _sc_edge_gather(z2, src, dst, batch):
    return _sc_kernels()[0](z2, src, dst, batch)


def _sc_scatter_relu(x, ea, pi, srcs, dsts, bnd):
    return _sc_kernels()[1](x, ea, pi, srcs, dsts, bnd)


def _sc_scatter_gate(z2, w, pi, srcs, dsts, bnd):
    return _sc_kernels()[2](z2, w, pi, srcs, dsts, bnd)


def _tc_edge_body(prod_ref, bsrc_ref, eattr_ref, lamr_ref, wq_ref, bq_ref,
                  wperm_ref, bo_ref, wpe_ref, bpe_ref, ea_ref, w_ref):
    eig = jnp.maximum(lamr_ref[...] * wq_ref[...] + bq_ref[...], 0.0)
    bs = bsrc_ref[0, 0, :]
    oh = (bs[:, None] == lax.broadcasted_iota(jnp.int32, (BE, G), 1))
    ee = jnp.dot(oh.astype(jnp.float32), eig, preferred_element_type=jnp.float32)
    prod = prod_ref[...]
    feats = []
    for q in range(Q):
        acc = prod[:, 0:HPE] * ee[:, q:q + 1]
        for p in range(1, P):
            acc = acc + prod[:, p * HPE:(p + 1) * HPE] * ee[:, p * Q + q:p * Q + q + 1]
        feats.append(acc)
    feat = jnp.concatenate(feats, axis=1)
    ea = (jnp.dot(feat, wperm_ref[...], preferred_element_type=jnp.float32)
          + bo_ref[...] + eattr_ref[...])
    ea_ref[...] = ea
    w = jax.nn.sigmoid(
        jnp.dot(ea, wpe_ref[...], preferred_element_type=jnp.float32) + bpe_ref[...])
    w_ref[...] = jnp.concatenate([w] * P, axis=1)


def _tc_edge(prod, bsrc3, edge_attr, lamr, wq, bq, wperm, bo, wpe, bpe):
    full = lambda s: pl.BlockSpec(s, lambda i: (0,) * len(s))
    return pl.pallas_call(
        _tc_edge_body,
        grid=(NBE,),
        in_specs=[
            pl.BlockSpec((BE, H), lambda i: (i, 0)),
            pl.BlockSpec((1, 1, BE), lambda i: (i, 0, 0)),
            pl.BlockSpec((BE, H), lambda i: (i, 0)),
            full((G, P * Q)),
            full((1, P * Q)),
            full((1, P * Q)),
            full((HPE * Q, H)),
            full((1, H)),
            full((H, HPE)),
            full((1, HPE)),
        ],
        out_specs=[
            pl.BlockSpec((BE, H), lambda i: (i, 0)),
            pl.BlockSpec((BE, H), lambda i: (i, 0)),
        ],
        out_shape=[
            jax.ShapeDtypeStruct((E, H), jnp.float32),
            jax.ShapeDtypeStruct((E, H), jnp.float32),
        ],
    )(prod, bsrc3, edge_attr, lamr, wq, bq, wperm, bo, wpe, bpe)


def _tc_node_x_body(last, x_ref, agg_ref, w1_ref, b1_ref,
                    g1_ref, bt1_ref, w2_ref, b2_ref, ng_ref, nb_ref, xo_ref):
    def bn(h, g, b):
        mu = jnp.mean(h, axis=0, keepdims=True)
        var = jnp.mean((h - mu) ** 2, axis=0, keepdims=True)
        return (h - mu) / jnp.sqrt(var + 1e-5) * g + b

    h = x_ref[...] + agg_ref[...]
    h = jnp.dot(h, w1_ref[...], preferred_element_type=jnp.float32) + b1_ref[...]
    h = jnp.maximum(bn(h, g1_ref[...], bt1_ref[...]), 0.0)
    xn = jnp.dot(h, w2_ref[...], preferred_element_type=jnp.float32) + b2_ref[...]
    xn = bn(xn, ng_ref[...], nb_ref[...])
    if not last:
        xn = jnp.maximum(xn, 0.0)
    xo_ref[...] = xn


def _tc_node_z_body(z2_ref, aggz_ref, plw_ref, plb_ref, zo_ref):
    hz = z2_ref[...] + aggz_ref[...]
    plw = plw_ref[...]
    cols = [
        jnp.dot(hz[:, p * HPE:(p + 1) * HPE], plw,
                preferred_element_type=jnp.float32) + plb_ref[...]
        for p in range(P)
    ]
    zo_ref[...] = jnp.concatenate(cols, axis=1)


def _tc_node(last, x, agg, z2, aggz,
             w1, b1, g1, bt1, w2, b2, plw, plb, ng, nb):
    xn = pl.pallas_call(
        functools.partial(_tc_node_x_body, last),
        out_shape=jax.ShapeDtypeStruct((N, H), jnp.float32),
    )(x, agg, w1, b1, g1, bt1, w2, b2, ng, nb)
    zn = pl.pallas_call(
        _tc_node_z_body,
        out_shape=jax.ShapeDtypeStruct((N, H), jnp.float32),
    )(z2, aggz, plw, plb)
    return xn, zn


def kernel(x, pe, Lambda, edge_index, edge_attr, batch, mlp_W1, mlp_b1,
           mlp_g1, mlp_bt1, mlp_W2, mlp_b2, pe_lin_W, pe_lin_b, pe_edge_W,
           pe_edge_b, enc_Wq, enc_bq, enc_out_W, enc_out_b, norm_g, norm_b):
    src = edge_index[0]
    dst = edge_index[1]
    pi = jnp.argsort(dst).astype(jnp.int32)
    dsts = dst[pi]
    srcs = src[pi]
    bnd = jnp.searchsorted(dsts, jnp.arange(NW + 1, dtype=jnp.int32) * ROWS_W,
                           ).astype(jnp.int32)
    bnd = jnp.concatenate([bnd, jnp.full((48 - NW - 1,), E, jnp.int32)])
    pad = jnp.zeros((GW,), jnp.int32)
    pi_p = jnp.concatenate([pi, pad])
    dsts_p = jnp.concatenate([dsts, pad])
    srcs_p = jnp.concatenate([srcs, pad])

    z2 = jnp.repeat(pe, HPE, axis=1)
    lamr = jnp.repeat(Lambda, Q, axis=1)

    x_cur = x
    for i in range(L):
        wq = jnp.tile(enc_Wq[i], P)[None, :]
        bq = jnp.tile(enc_bq[i], P)[None, :]
        wperm = enc_out_W[i].reshape(HPE, Q, H).transpose(1, 0, 2).reshape(HPE * Q, H)
        bo = enc_out_b[i][None, :]
        wpe = pe_edge_W[i]
        bpe = pe_edge_b[i][None, :]

        prod, bsrc = _sc_edge_gather(z2, src, dst, batch)
        bsrc3 = bsrc.reshape(NBE, 1, BE)
        ea, w = _tc_edge(prod, bsrc3, edge_attr, lamr, wq, bq, wperm, bo, wpe, bpe)
        agg = _sc_scatter_relu(x_cur, ea, pi_p, srcs_p, dsts_p, bnd)
        aggz = _sc_scatter_gate(z2, w, pi_p, srcs_p, dsts_p, bnd)
        x_cur, z2 = _tc_node(
            i == L - 1, x_cur, agg[:N], z2, aggz[:N],
            mlp_W1[i], mlp_b1[i][None, :], mlp_g1[i][None, :],
            mlp_bt1[i][None, :], mlp_W2[i], mlp_b2[i][None, :],
            pe_lin_W[i], pe_lin_b[i][None, :],
            norm_g[i][None, :], norm_b[i][None, :])

    z_out = z2.reshape(N, P, HPE).transpose(0, 2, 1)
    return x_cur, z_out


# speedup vs baseline: 8.9817x; 1.0361x over previous
import functools

import jax
import jax.numpy as jnp
from jax import lax
from jax.experimental import pallas as pl
from jax.experimental.pallas import tpu as pltpu
from jax.experimental.pallas import tpu_sc as plsc

N = 10000
H = 128
HPE = 16
P = 8
Q = 4
G = 16
E = 160000
L = 3

W = 128
NBLK = E // W
NW = 32
NS = 16
N2 = 10240
ROWS_W = N2 // NW
GW = 128

BE = 1280
NBE = E // BE


def _nblocks_for(wid, nworkers):
    full = NBLK // nworkers
    extra = NBLK % nworkers
    return jnp.where(wid < extra, full + 1, full).astype(jnp.int32)


def _sc_edge_gather_body(z2_hbm, src_hbm, dst_hbm, batch_hbm,
                         prod_hbm, bsrc_hbm,
                         src0, src1, dst0, dst1, zs0, zs1, zd0, zd1,
                         prod0, prod1, bs0, bs1,
                         si0, si1, sp0, sp1, so0, so1):
    wid = (lax.axis_index("s") * 2 + lax.axis_index("c")).astype(jnp.int32)
    nb = _nblocks_for(wid, NW)
    bufs = ((src0, dst0, si0, zs0, zd0, bs0, sp0, prod0, so0),
            (src1, dst1, si1, zs1, zd1, bs1, sp1, prod1, so1))

    def base_of(g):
        return (wid + g * NW) * W

    def issue_idx(k, g):
        src_v, dst_v, si = bufs[k][:3]
        e0 = base_of(g)
        pltpu.async_copy(src_hbm.at[pl.ds(e0, W)], src_v, si)
        pltpu.async_copy(dst_hbm.at[pl.ds(e0, W)], dst_v, si)

    def wait_idx(k):
        src_v, dst_v, si = bufs[k][:3]
        pltpu.make_async_copy(src_hbm.at[pl.ds(0, W)], src_v, si).wait()
        pltpu.make_async_copy(src_hbm.at[pl.ds(0, W)], dst_v, si).wait()

    def issue_pay(k):
        src_v, dst_v, _, zs_v, zd_v, bs_v, sp = bufs[k][:7]
        pltpu.async_copy(z2_hbm.at[src_v], zs_v, sp)
        pltpu.async_copy(z2_hbm.at[dst_v], zd_v, sp)
        pltpu.async_copy(batch_hbm.at[src_v], bs_v, sp)

    def wait_pay(k):
        _, _, _, zs_v, zd_v, bs_v, sp = bufs[k][:7]
        pltpu.make_async_copy(z2_hbm.at[pl.ds(0, W)], zs_v, sp).wait()
        pltpu.make_async_copy(z2_hbm.at[pl.ds(0, W)], zd_v, sp).wait()
        pltpu.make_async_copy(src_hbm.at[pl.ds(0, W)], bs_v, sp).wait()

    def compute(k):
        zs_v, zd_v = bufs[k][3], bufs[k][4]
        prod_v = bufs[k][7]

        def edge_body(e, _):
            for c in range(H // 16):
                sl = pl.ds(c * 16, 16)
                prod_v[e, sl] = zs_v[e, sl] * zd_v[e, sl]
            return 0

        lax.fori_loop(0, W, edge_body, 0)

    def issue_out(k, g):
        bs_v, prod_v, so = bufs[k][5], bufs[k][7], bufs[k][8]
        e0 = base_of(g)
        pltpu.async_copy(prod_v, prod_hbm.at[pl.ds(e0, W)], so)
        pltpu.async_copy(bs_v, bsrc_hbm.at[pl.ds(e0, W)], so)

    def wait_out(k):
        bs_v, prod_v, so = bufs[k][5], bufs[k][7], bufs[k][8]
        pltpu.make_async_copy(prod_v, prod_hbm.at[pl.ds(0, W)], so).wait()
        pltpu.make_async_copy(bs_v, bsrc_hbm.at[pl.ds(0, W)], so).wait()

    @pl.when(nb >= 1)
    def _():
        issue_idx(0, 0)

    def phase(k, g):
        nk = 1 - k

        @pl.when(g < nb)
        def _():
            wait_idx(k)

            @pl.when(g >= 2)
            def _():
                wait_out(k)

            issue_pay(k)

        @pl.when((g >= 1) & (g <= nb))
        def _():
            wait_pay(nk)
            compute(nk)
            issue_out(nk, g - 1)

        @pl.when(g + 1 < nb)
        def _():
            issue_idx(nk, g + 1)

    def pair(g2, _):
        phase(0, g2 * 2)
        phase(1, g2 * 2 + 1)
        return 0

    lax.fori_loop(0, (nb + 2) // 2, pair, 0)

    @pl.when((nb >= 1) & ((nb - 1) % 2 == 0))
    def _():
        wait_out(0)

    @pl.when((nb >= 1) & ((nb - 1) % 2 == 1))
    def _():
        wait_out(1)

    @pl.when((nb >= 2) & (nb % 2 == 0))
    def _():
        wait_out(0)

    @pl.when((nb >= 2) & (nb % 2 == 1))
    def _():
        wait_out(1)


def _make_sc_scatter_body(gate):
    def body(table_hbm, pay_hbm, pi_hbm, srcs_hbm, dsts_hbm, bnd_hbm, out_hbm,
             bnd_v, ids0, ids1, srw0, srw1, dsw0, dsw1, dhold,
             pay0, pay1, gat0, gat1, acc_v,
             si0, si1, sp0, sp1):
        core = lax.axis_index("c").astype(jnp.int32)
        sid = lax.axis_index("s").astype(jnp.int32)
        w = sid * 2 + core
        lo = w * ROWS_W

        pltpu.sync_copy(bnd_hbm, bnd_v)
        s0 = bnd_v[pl.ds(w, 16)][0]
        s1 = bnd_v[pl.ds(w + 1, 16)][0]

        def zrow(r, _):
            for c in range(H // 16):
                acc_v[r, pl.ds(c * 16, 16)] = jnp.zeros((16,), jnp.float32)
            return 0

        lax.fori_loop(0, ROWS_W, zrow, 0)

        sA = pl.multiple_of((s0 // GW) * GW, GW)
        nwin = (s1 - sA + GW - 1) // GW
        bufs = ((ids0, srw0, dsw0, si0, pay0, gat0, sp0),
                (ids1, srw1, dsw1, si1, pay1, gat1, sp1))

        def issue_idx(k, g):
            ids, srw, dsw, si = bufs[k][:4]
            base = pl.multiple_of(sA + g * GW, GW)
            pltpu.async_copy(pi_hbm.at[pl.ds(base, GW)], ids, si)
            pltpu.async_copy(srcs_hbm.at[pl.ds(base, GW)], srw, si)
            pltpu.async_copy(dsts_hbm.at[pl.ds(base, GW)], dsw, si)

        def wait_idx(k):
            ids, srw, dsw, si = bufs[k][:4]
            pltpu.make_async_copy(pi_hbm.at[pl.ds(0, GW)], ids, si).wait()
            pltpu.make_async_copy(pi_hbm.at[pl.ds(0, GW)], srw, si).wait()
            pltpu.make_async_copy(pi_hbm.at[pl.ds(0, GW)], dsw, si).wait()

        def issue_pay(k):
            ids, srw, _, _, pay, gat, sp = bufs[k]
            pltpu.async_copy(pay_hbm.at[ids], pay, sp)
            pltpu.async_copy(table_hbm.at[srw], gat, sp)

        def wait_pay(k):
            _, _, _, _, pay, gat, sp = bufs[k]
            pltpu.make_async_copy(pay_hbm.at[pl.ds(0, GW)], pay, sp).wait()
            pltpu.make_async_copy(pay_hbm.at[pl.ds(0, GW)], gat, sp).wait()

        def compute(k, g):
            _, _, _, _, pay, gat, _ = bufs[k]
            base = sA + g * GW

            def chunk(c, _):
                dv = dhold[pl.ds(c * 16, 16)]
                for jj in range(16):
                    gg = base + c * 16 + jj
                    r = dv[jj] - lo

                    @pl.when((gg >= s0) & (gg < s1))
                    def _():
                        def upd(cc, _):
                            e = c * 16 + jj
                            sl = pl.ds(cc * 16, 16)
                            if gate:
                                acc_v[r, sl] = (acc_v[r, sl]
                                                + gat[e, sl] * pay[e, sl])
                            else:
                                acc_v[r, sl] = acc_v[r, sl] + jnp.maximum(
                                    gat[e, sl] + pay[e, sl], 0.0)
                            return 0

                        lax.fori_loop(0, H // 16, upd, 0)
                return 0

            lax.fori_loop(0, GW // 16, chunk, 0)

        def hold_dst(k):
            dsw = bufs[k][2]
            for c in range(GW // 16):
                sl = pl.ds(c * 16, 16)
                dhold[sl] = dsw[sl]

        @pl.when(nwin >= 1)
        def _():
            issue_idx(0, 0)

        def phase(k, g):
            nk = 1 - k

            @pl.when(g < nwin)
            def _():
                wait_idx(k)
                issue_pay(k)

            @pl.when((g >= 1) & (g <= nwin))
            def _():
                wait_pay(nk)
                hold_dst(nk)

            @pl.when(g + 1 < nwin)
            def _():
                issue_idx(nk, g + 1)

            @pl.when((g >= 1) & (g <= nwin))
            def _():
                compute(nk, g - 1)

        def pair(g2, _):
            phase(0, g2 * 2)
            phase(1, g2 * 2 + 1)
            return 0

        lax.fori_loop(0, (nwin + 2) // 2, pair, 0)

        pltpu.sync_copy(acc_v, out_hbm.at[pl.ds(lo, ROWS_W)])

    return body


@functools.lru_cache(maxsize=None)
def _sc_kernels():
    mesh = plsc.VectorSubcoreMesh(
        core_axis_name="c", subcore_axis_name="s", num_cores=2, num_subcores=16)
    edge_gather = pl.kernel(
        _sc_edge_gather_body,
        out_type=(
            jax.ShapeDtypeStruct((E, H), jnp.float32),
            jax.ShapeDtypeStruct((E,), jnp.int32),
        ),
        mesh=mesh,
        scratch_types=[
            pltpu.VMEM((W,), jnp.int32),
            pltpu.VMEM((W,), jnp.int32),
            pltpu.VMEM((W,), jnp.int32),
            pltpu.VMEM((W,), jnp.int32),
            pltpu.VMEM((W, H), jnp.float32),
            pltpu.VMEM((W, H), jnp.float32),
            pltpu.VMEM((W, H), jnp.float32),
            pltpu.VMEM((W, H), jnp.float32),
            pltpu.VMEM((W, H), jnp.float32),
            pltpu.VMEM((W, H), jnp.float32),
            pltpu.VMEM((W,), jnp.int32),
            pltpu.VMEM((W,), jnp.int32),
            pltpu.SemaphoreType.DMA,
            pltpu.SemaphoreType.DMA,
            pltpu.SemaphoreType.DMA,
            pltpu.SemaphoreType.DMA,
            pltpu.SemaphoreType.DMA,
            pltpu.SemaphoreType.DMA,
        ],
    )
    def make_scatter(gate):
        payw = H
        return pl.kernel(
            _make_sc_scatter_body(gate),
            out_type=jax.ShapeDtypeStruct((N2, H), jnp.float32),
            mesh=mesh,
            scratch_types=[
                pltpu.VMEM((48,), jnp.int32),
                pltpu.VMEM((GW,), jnp.int32),
                pltpu.VMEM((GW,), jnp.int32),
                pltpu.VMEM((GW,), jnp.int32),
                pltpu.VMEM((GW,), jnp.int32),
                pltpu.VMEM((GW,), jnp.int32),
                pltpu.VMEM((GW,), jnp.int32),
                pltpu.VMEM((GW,), jnp.int32),
                pltpu.VMEM((GW, payw), jnp.float32),
                pltpu.VMEM((GW, payw), jnp.float32),
                pltpu.VMEM((GW, H), jnp.float32),
                pltpu.VMEM((GW, H), jnp.float32),
                pltpu.VMEM((ROWS_W, H), jnp.float32),
                pltpu.SemaphoreType.DMA,
                pltpu.SemaphoreType.DMA,
                pltpu.SemaphoreType.DMA,
                pltpu.SemaphoreType.DMA,
            ],
        )

    return edge_gather, make_scatter(False), make_scatter(True)


def _sc_edge_gather(z2, src, dst, batch):
    return _sc_kernels()[0](z2, src, dst, batch)


def _sc_scatter_relu(x, ea, pi, srcs, dsts, bnd):
    return _sc_kernels()[1](x, ea, pi, srcs, dsts, bnd)


def _sc_scatter_gate(z2, w, pi, srcs, dsts, bnd):
    return _sc_kernels()[2](z2, w, pi, srcs, dsts, bnd)


def _tc_edge_body(prod_ref, bsrc_ref, eattr_ref, lamr_ref, wq_ref, bq_ref,
                  wperm_ref, bo_ref, wpe_ref, bpe_ref, ea_ref, w_ref):
    eig = jnp.maximum(lamr_ref[...] * wq_ref[...] + bq_ref[...], 0.0)
    bs = bsrc_ref[0, 0, :]
    oh = (bs[:, None] == lax.broadcasted_iota(jnp.int32, (BE, G), 1))
    ee = jnp.dot(oh.astype(jnp.float32), eig, preferred_element_type=jnp.float32)
    prod = prod_ref[...]
    feats = []
    for q in range(Q):
        acc = prod[:, 0:HPE] * ee[:, q:q + 1]
        for p in range(1, P):
            acc = acc + prod[:, p * HPE:(p + 1) * HPE] * ee[:, p * Q + q:p * Q + q + 1]
        feats.append(acc)
    feat = jnp.concatenate(feats, axis=1)
    ea = (jnp.dot(feat, wperm_ref[...], preferred_element_type=jnp.float32)
          + bo_ref[...] + eattr_ref[...])
    ea_ref[...] = ea
    w = jax.nn.sigmoid(
        jnp.dot(ea, wpe_ref[...], preferred_element_type=jnp.float32) + bpe_ref[...])
    w_ref[...] = jnp.concatenate([w] * P, axis=1)


def _tc_edge(prod, bsrc3, edge_attr, lamr, wq, bq, wperm, bo, wpe, bpe):
    full = lambda s: pl.BlockSpec(s, lambda i: (0,) * len(s))
    return pl.pallas_call(
        _tc_edge_body,
        grid=(NBE,),
        in_specs=[
            pl.BlockSpec((BE, H), lambda i: (i, 0)),
            pl.BlockSpec((1, 1, BE), lambda i: (i, 0, 0)),
            pl.BlockSpec((BE, H), lambda i: (i, 0)),
            full((G, P * Q)),
            full((1, P * Q)),
            full((1, P * Q)),
            full((HPE * Q, H)),
            full((1, H)),
            full((H, HPE)),
            full((1, HPE)),
        ],
        out_specs=[
            pl.BlockSpec((BE, H), lambda i: (i, 0)),
            pl.BlockSpec((BE, H), lambda i: (i, 0)),
        ],
        out_shape=[
            jax.ShapeDtypeStruct((E, H), jnp.float32),
            jax.ShapeDtypeStruct((E, H), jnp.float32),
        ],
    )(prod, bsrc3, edge_attr, lamr, wq, bq, wperm, bo, wpe, bpe)


def _tc_node_x_body(last, x_ref, agg_ref, w1_ref, b1_ref,
                    g1_ref, bt1_ref, w2_ref, b2_ref, ng_ref, nb_ref, xo_ref):
    def bn(h, g, b):
        mu = jnp.mean(h, axis=0, keepdims=True)
        var = jnp.mean((h - mu) ** 2, axis=0, keepdims=True)
        return (h - mu) / jnp.sqrt(var + 1e-5) * g + b

    h = x_ref[...] + agg_ref[...]
    h = jnp.dot(h, w1_ref[...], preferred_element_type=jnp.float32) + b1_ref[...]
    h = jnp.maximum(bn(h, g1_ref[...], bt1_ref[...]), 0.0)
    xn = jnp.dot(h, w2_ref[...], preferred_element_type=jnp.float32) + b2_ref[...]
    xn = bn(xn, ng_ref[...], nb_ref[...])
    if not last:
        xn = jnp.maximum(xn, 0.0)
    xo_ref[...] = xn


def _tc_node_z_body(z2_ref, aggz_ref, plw_ref, plb_ref, zo_ref):
    hz = z2_ref[...] + aggz_ref[...]
    plw = plw_ref[...]
    cols = [
        jnp.dot(hz[:, p * HPE:(p + 1) * HPE], plw,
                preferred_element_type=jnp.float32) + plb_ref[...]
        for p in range(P)
    ]
    zo_ref[...] = jnp.concatenate(cols, axis=1)


def _tc_node(last, x, agg, z2, aggz,
             w1, b1, g1, bt1, w2, b2, plw, plb, ng, nb):
    xn = pl.pallas_call(
        functools.partial(_tc_node_x_body, last),
        out_shape=jax.ShapeDtypeStruct((N, H), jnp.float32),
    )(x, agg, w1, b1, g1, bt1, w2, b2, ng, nb)
    zn = pl.pallas_call(
        _tc_node_z_body,
        out_shape=jax.ShapeDtypeStruct((N, H), jnp.float32),
    )(z2, aggz, plw, plb)
    return xn, zn


def kernel(x, pe, Lambda, edge_index, edge_attr, batch, mlp_W1, mlp_b1,
           mlp_g1, mlp_bt1, mlp_W2, mlp_b2, pe_lin_W, pe_lin_b, pe_edge_W,
           pe_edge_b, enc_Wq, enc_bq, enc_out_W, enc_out_b, norm_g, norm_b):
    src = edge_index[0]
    dst = edge_index[1]
    pi = jnp.argsort(dst).astype(jnp.int32)
    dsts = dst[pi]
    srcs = src[pi]
    bnd = jnp.searchsorted(dsts, jnp.arange(NW + 1, dtype=jnp.int32) * ROWS_W,
                           ).astype(jnp.int32)
    bnd = jnp.concatenate([bnd, jnp.full((48 - NW - 1,), E, jnp.int32)])
    pad = jnp.zeros((GW,), jnp.int32)
    pi_p = jnp.concatenate([pi, pad])
    dsts_p = jnp.concatenate([dsts, pad])
    srcs_p = jnp.concatenate([srcs, pad])

    z2 = jnp.repeat(pe, HPE, axis=1)
    lamr = jnp.repeat(Lambda, Q, axis=1)

    x_cur = x
    for i in range(L):
        wq = jnp.tile(enc_Wq[i], P)[None, :]
        bq = jnp.tile(enc_bq[i], P)[None, :]
        wperm = enc_out_W[i].reshape(HPE, Q, H).transpose(1, 0, 2).reshape(HPE * Q, H)
        bo = enc_out_b[i][None, :]
        wpe = pe_edge_W[i]
        bpe = pe_edge_b[i][None, :]

        prod, bsrc = _sc_edge_gather(z2, src, dst, batch)
        bsrc3 = bsrc.reshape(NBE, 1, BE)
        ea, w = _tc_edge(prod, bsrc3, edge_attr, lamr, wq, bq, wperm, bo, wpe, bpe)
        agg = _sc_scatter_relu(x_cur, ea, pi_p, srcs_p, dsts_p, bnd)
        aggz = _sc_scatter_gate(z2, w, pi_p, srcs_p, dsts_p, bnd)
        x_cur, z2 = _tc_node(
            i == L - 1, x_cur, agg[:N], z2, aggz[:N],
            mlp_W1[i], mlp_b1[i][None, :], mlp_g1[i][None, :],
            mlp_bt1[i][None, :], mlp_W2[i], mlp_b2[i][None, :],
            pe_lin_W[i], pe_lin_b[i][None, :],
            norm_g[i][None, :], norm_b[i][None, :])

    z_out = z2.reshape(N, P, HPE).transpose(0, 2, 1)
    return x_cur, z_out
